# baseline (speedup 1.0000x reference)
import functools

import jax
import jax.numpy as jnp
from jax import lax
from jax.experimental import pallas as pl
from jax.experimental.pallas import tpu as pltpu
from jax.experimental.pallas import tpu_sc as plsc

B = 16384
D = 64
NC = 2
NS = 16
NW = NC * NS
EPW = B // NW
RPW = 2 * EPW
CH = 32
OPC = CH // 2
NCH = RPW // CH

_mesh = plsc.VectorSubcoreMesh(core_axis_name="c", subcore_axis_name="s")


@functools.partial(
    pl.kernel,
    mesh=_mesh,
    out_type=jax.ShapeDtypeStruct((B, 2 * D), jnp.float32),
    scratch_types=[
        pltpu.VMEM((EPW,), jnp.int32),
        pltpu.VMEM((EPW,), jnp.int32),
        pltpu.VMEM((CH // 8, 8, D), jnp.float32),
        pltpu.VMEM((CH // 8, 8, D), jnp.float32),
        pltpu.VMEM((OPC, 2 * D), jnp.float32),
        pltpu.VMEM((OPC, 2 * D), jnp.float32),
        pltpu.SemaphoreType.DMA,
        pltpu.SemaphoreType.DMA,
        pltpu.SemaphoreType.DMA,
        pltpu.SemaphoreType.DMA,
    ],
)
def _edge_gather(src_hbm, dst_hbm, table_hbm, out_hbm,
                 srcv, dstv, row0, row1, ob0, ob1, sem0, sem1, osem0, osem1):
    wid = lax.axis_index("s") * NC + lax.axis_index("c")
    eb = wid * EPW

    pltpu.sync_copy(src_hbm.at[pl.ds(eb, EPW)], srcv)
    pltpu.sync_copy(dst_hbm.at[pl.ds(eb, EPW)], dstv)

    def issue(c, buf, sem):
        for g in range(OPC // 16):
            vs = srcv[pl.ds(c * OPC + g * 16, 16)]
            vd = dstv[pl.ds(c * OPC + g * 16, 16)]
            ts = lax.shift_right_logical(vs, 3)
            td = lax.shift_right_logical(vd, 3)
            rs = lax.bitwise_and(vs, 7)
            rd = lax.bitwise_and(vd, 7)
            for l in range(16):
                j = 2 * (g * 16 + l)
                pltpu.async_copy(
                    table_hbm.at[ts[l], pl.ds(rs[l], 1), :],
                    buf.at[j // 8, pl.ds(j % 8, 1), :],
                    sem,
                )
                pltpu.async_copy(
                    table_hbm.at[td[l], pl.ds(rd[l], 1), :],
                    buf.at[(j + 1) // 8, pl.ds((j + 1) % 8, 1), :],
                    sem,
                )

    def drain(buf, sem):
        pltpu.make_async_copy(
            table_hbm.at[pl.ds(0, CH // 8)],
            buf,
            sem,
        ).wait()

    def out_slab(c):
        return out_hbm.at[pl.ds(eb + c * OPC, OPC)]

    def assemble(buf, ob):
        for r in range(OPC):
            a, b = 2 * r, 2 * r + 1
            for j in range(D // 16):
                ob[r, pl.ds(16 * j, 16)] = buf[a // 8, a % 8, pl.ds(16 * j, 16)]
                ob[r, pl.ds(D + 16 * j, 16)] = (
                    buf[b // 8, b % 8, pl.ds(16 * j, 16)])

    issue(0, row0, sem0)

    @pl.loop(0, NCH // 2)
    def pipeline(i):
        c0 = 2 * i
        issue(c0 + 1, row1, sem1)
        drain(row0, sem0)

        @pl.when(i > 0)
        def _():
            pltpu.make_async_copy(ob0, out_slab(c0), osem0).wait()

        assemble(row0, ob0)
        pltpu.async_copy(ob0, out_slab(c0), osem0)

        @pl.when(i < NCH // 2 - 1)
        def _():
            issue(c0 + 2, row0, sem0)

        drain(row1, sem1)

        @pl.when(i > 0)
        def _():
            pltpu.make_async_copy(ob1, out_slab(c0 + 1), osem1).wait()

        assemble(row1, ob1)
        pltpu.async_copy(ob1, out_slab(c0 + 1), osem1)

    pltpu.make_async_copy(ob0, out_slab(0), osem0).wait()
    pltpu.make_async_copy(ob1, out_slab(1), osem1).wait()


def kernel(source_node_input, destination_node_input, embedding_table):
    return _edge_gather(
        source_node_input.reshape(B),
        destination_node_input.reshape(B),
        embedding_table.reshape(125000, 8, D),
    )


# speedup vs baseline: 1.0030x; 1.0030x over previous
import functools

import jax
import jax.numpy as jnp
from jax import lax
from jax.experimental import pallas as pl
from jax.experimental.pallas import tpu as pltpu
from jax.experimental.pallas import tpu_sc as plsc

B = 16384
D = 64
NC = 2
NS = 16
NW = NC * NS
EPW = B // NW
RPW = 2 * EPW
CH = 32
OPC = CH // 2
NCH = RPW // CH

_mesh = plsc.VectorSubcoreMesh(core_axis_name="c", subcore_axis_name="s")


@functools.partial(
    pl.kernel,
    mesh=_mesh,
    out_type=jax.ShapeDtypeStruct((B, 2 * D), jnp.float32),
    scratch_types=[
        pltpu.VMEM((EPW,), jnp.int32),
        pltpu.VMEM((EPW,), jnp.int32),
        pltpu.VMEM((CH // 8, 8, D), jnp.float32),
        pltpu.VMEM((CH // 8, 8, D), jnp.float32),
        pltpu.VMEM((OPC, 2 * D), jnp.float32),
        pltpu.VMEM((OPC, 2 * D), jnp.float32),
        pltpu.SemaphoreType.DMA,
        pltpu.SemaphoreType.DMA,
        pltpu.SemaphoreType.DMA,
        pltpu.SemaphoreType.DMA,
    ],
)
def _edge_gather(src_hbm, dst_hbm, table_hbm, out_hbm,
                 srcv, dstv, row0, row1, ob0, ob1, sem0, sem1, osem0, osem1):
    wid = lax.axis_index("s") * NC + lax.axis_index("c")
    eb = wid * EPW

    pltpu.sync_copy(src_hbm.at[pl.ds(eb, EPW)], srcv)
    pltpu.sync_copy(dst_hbm.at[pl.ds(eb, EPW)], dstv)

    def issue(c, buf, sem):
        for g in range(OPC // 16):
            vs = srcv[pl.ds(c * OPC + g * 16, 16)]
            vd = dstv[pl.ds(c * OPC + g * 16, 16)]
            for l in range(16):
                j = 2 * (g * 16 + l)
                sv = vs[l]
                dv = vd[l]
                pltpu.async_copy(
                    table_hbm.at[lax.shift_right_logical(sv, 3),
                                 pl.ds(lax.bitwise_and(sv, 7), 1), :],
                    buf.at[j // 8, pl.ds(j % 8, 1), :],
                    sem,
                )
                pltpu.async_copy(
                    table_hbm.at[lax.shift_right_logical(dv, 3),
                                 pl.ds(lax.bitwise_and(dv, 7), 1), :],
                    buf.at[(j + 1) // 8, pl.ds((j + 1) % 8, 1), :],
                    sem,
                )

    def drain(buf, sem):
        pltpu.make_async_copy(
            table_hbm.at[pl.ds(0, CH // 8)],
            buf,
            sem,
        ).wait()

    def out_slab(c):
        return out_hbm.at[pl.ds(eb + c * OPC, OPC)]

    def assemble(buf, ob):
        for r in range(OPC):
            a, b = 2 * r, 2 * r + 1
            for j in range(D // 16):
                ob[r, pl.ds(16 * j, 16)] = buf[a // 8, a % 8, pl.ds(16 * j, 16)]
                ob[r, pl.ds(D + 16 * j, 16)] = (
                    buf[b // 8, b % 8, pl.ds(16 * j, 16)])

    issue(0, row0, sem0)

    @pl.loop(0, NCH // 2)
    def pipeline(i):
        c0 = 2 * i
        issue(c0 + 1, row1, sem1)
        drain(row0, sem0)

        @pl.when(i > 0)
        def _():
            pltpu.make_async_copy(ob0, out_slab(c0), osem0).wait()

        assemble(row0, ob0)
        pltpu.async_copy(ob0, out_slab(c0), osem0)

        @pl.when(i < NCH // 2 - 1)
        def _():
            issue(c0 + 2, row0, sem0)

        drain(row1, sem1)

        @pl.when(i > 0)
        def _():
            pltpu.make_async_copy(ob1, out_slab(c0 + 1), osem1).wait()

        assemble(row1, ob1)
        pltpu.async_copy(ob1, out_slab(c0 + 1), osem1)

    pltpu.make_async_copy(ob0, out_slab(0), osem0).wait()
    pltpu.make_async_copy(ob1, out_slab(1), osem1).wait()


def kernel(source_node_input, destination_node_input, embedding_table):
    return _edge_gather(
        source_node_input.reshape(B),
        destination_node_input.reshape(B),
        embedding_table.reshape(125000, 8, D),
    )
